# baseline (speedup 1.0000x reference)
import functools

import jax
import jax.numpy as jnp
from jax import lax
from jax.experimental import pallas as pl
from jax.experimental.pallas import tpu as pltpu
from jax.experimental.pallas import tpu_sc as plsc

N_NODES = 10000
N_EDGES = 320000
D = 128
D_EDGE = 16
NEG_SLOPE = 0.2

NODE_BLK = 1000
EDGE_BLK = 4000

N_WORKERS = 32
PER_CORE = N_EDGES // 2
PER_W = N_EDGES // N_WORKERS
C = 40
N_CHUNK = PER_W // C
NBUF = 2
N_PAD = 10240
RPT = N_PAD // 16
LANES = 16


N_NODE_BLKS = N_NODES // NODE_BLK


def _tc_front_body(ea_ref, w3_ref, batt_ref, x_ref, wfc_ref, bfc_ref,
                   w1_ref, w2_ref, e_ref, a_ref, tbh_ref):
    i = pl.program_id(0)
    e = jnp.dot(ea_ref[...], w3_ref[...], preferred_element_type=jnp.float32)
    e_ref[...] = e + batt_ref[...]

    @pl.when(i < N_NODE_BLKS)
    def _():
        h = jnp.dot(x_ref[...], wfc_ref[...], preferred_element_type=jnp.float32)
        h = h + bfc_ref[...]
        a_ref[...] = jnp.dot(h, w1_ref[...], preferred_element_type=jnp.float32)
        b = jnp.dot(h, w2_ref[...], preferred_element_type=jnp.float32)
        tbh_ref[...] = jnp.concatenate([b, h], axis=1)


def _node_blk_map(i):
    return (jnp.minimum(i, N_NODE_BLKS - 1), 0)


_tc_front = pl.pallas_call(
    _tc_front_body,
    grid=(N_EDGES // EDGE_BLK,),
    in_specs=[
        pl.BlockSpec((EDGE_BLK, D_EDGE), lambda i: (i, 0)),
        pl.BlockSpec((D_EDGE, D), lambda i: (0, 0)),
        pl.BlockSpec((1, D), lambda i: (0, 0)),
        pl.BlockSpec((NODE_BLK, D), _node_blk_map),
        pl.BlockSpec((D, D), lambda i: (0, 0)),
        pl.BlockSpec((1, D), lambda i: (0, 0)),
        pl.BlockSpec((D, D), lambda i: (0, 0)),
        pl.BlockSpec((D, D), lambda i: (0, 0)),
    ],
    out_specs=[
        pl.BlockSpec((EDGE_BLK, D), lambda i: (i, 0)),
        pl.BlockSpec((NODE_BLK, D), _node_blk_map),
        pl.BlockSpec((NODE_BLK, 2 * D), _node_blk_map),
    ],
    out_shape=[
        jax.ShapeDtypeStruct((N_EDGES, D), jnp.float32),
        jax.ShapeDtypeStruct((N_NODES, D), jnp.float32),
        jax.ShapeDtypeStruct((N_NODES, 2 * D), jnp.float32),
    ],
)


def _tc_sum_body(p_ref, o_ref):
    o_ref[...] = p_ref[0] + p_ref[1]


_tc_sum = pl.pallas_call(
    _tc_sum_body,
    grid=(N_NODES // NODE_BLK,),
    in_specs=[pl.BlockSpec((2, NODE_BLK, D), lambda i: (0, i, 0))],

    out_specs=pl.BlockSpec((NODE_BLK, D), lambda i: (i, 0)),
    out_shape=jax.ShapeDtypeStruct((N_NODES, D), jnp.float32),
)


_sc_mesh = plsc.VectorSubcoreMesh(core_axis_name="c", subcore_axis_name="s")


@functools.partial(
    pl.kernel,
    mesh=_sc_mesh,
    out_type=jax.ShapeDtypeStruct((2, N_PAD, D), jnp.float32),
    scratch_types=[
        pltpu.VMEM((NBUF, 2, C), jnp.int32),
        pltpu.VMEM((NBUF, C, D), jnp.float32),
        pltpu.VMEM((NBUF, C, 2 * D), jnp.float32),
        pltpu.VMEM((NBUF, C, D), jnp.float32),
        pltpu.VMEM_SHARED((N_PAD, D), jnp.float32),
        pltpu.SemaphoreType.DMA,
        pltpu.SemaphoreType.DMA,
    ],
)
def _sc_edges(a_hbm, tbh_hbm, e_hbm, rc_hbm, zero_hbm, out_hbm,
              idx, ga, gtbh, ge, acc, sem0, sem1):
    c = lax.axis_index("c")
    s = lax.axis_index("s")
    sems = (sem0, sem1)

    rowbase = s * RPT
    pltpu.sync_copy(zero_hbm.at[pl.ds(rowbase, RPT)],
                    acc.at[pl.ds(rowbase, RPT)])

    ebase = c * PER_CORE + s * PER_W
    cbase = ebase // C

    def _issue(b, k):
        pltpu.sync_copy(rc_hbm.at[cbase + k], idx.at[b])
        pltpu.async_copy(a_hbm.at[idx.at[b, 0]], ga.at[b], sems[b])
        pltpu.async_copy(tbh_hbm.at[idx.at[b, 1]], gtbh.at[b], sems[b])
        pltpu.async_copy(e_hbm.at[pl.ds(ebase + k * C, C)], ge.at[b], sems[b])

    for b in range(NBUF):
        _issue(b, b)
    plsc.subcore_barrier()

    def _pair(t, carry):
        for b in range(NBUF):
            k = t * NBUF + b
            off = ebase + k * C
            pltpu.make_async_copy(a_hbm.at[idx.at[b, 0]], ga.at[b], sems[b]).wait()
            pltpu.make_async_copy(tbh_hbm.at[idx.at[b, 1]], gtbh.at[b], sems[b]).wait()
            pltpu.make_async_copy(e_hbm.at[pl.ds(off, C)], ge.at[b], sems[b]).wait()

            @plsc.parallel_loop(0, C, unroll=16)
            def _edge(i):
                for j in range(D // LANES):
                    sl = pl.ds(j * LANES, LANES)
                    sv = ga[b, i, sl] + gtbh[b, i, sl] + ge[b, i, sl]
                    alpha = jnp.maximum(sv, sv * jnp.float32(NEG_SLOPE))
                    ga[b, i, sl] = gtbh[b, i, pl.ds(D + j * LANES, LANES)] * alpha

            pltpu.sync_copy(ga.at[b], acc.at[idx.at[b, 0]], add=True)

            @pl.when(k + NBUF < N_CHUNK)
            def _():
                _issue(b, k + NBUF)
        return carry

    lax.fori_loop(0, N_CHUNK // NBUF, _pair, 0)
    plsc.subcore_barrier()

    pltpu.sync_copy(acc.at[pl.ds(rowbase, RPT)],
                    out_hbm.at[c, pl.ds(rowbase, RPT)])


@jax.jit
def kernel(x, edge_index, edge_attr, W_fc, b_fc, W_att, b_att):
    row = edge_index[0].astype(jnp.int32)
    col = edge_index[1].astype(jnp.int32)
    W1 = W_att[:D]
    W2 = W_att[D:2 * D]
    W3 = W_att[2 * D:]
    E, A, TBH = _tc_front(edge_attr, W3, b_att.reshape(1, D),
                          x, W_fc, b_fc.reshape(1, D), W1, W2)
    zeros = jnp.zeros((N_PAD, D), jnp.float32)
    rc = jnp.stack([row.reshape(-1, C), col.reshape(-1, C)], axis=1)
    partials = _sc_edges(A, TBH, E, rc, zeros)
    return _tc_sum(partials)


# speedup vs baseline: 1.1504x; 1.1504x over previous
import functools

import jax
import jax.numpy as jnp
from jax import lax
from jax.experimental import pallas as pl
from jax.experimental.pallas import tpu as pltpu
from jax.experimental.pallas import tpu_sc as plsc

N_NODES = 10000
N_EDGES = 320000
D = 128
D_EDGE = 16
NEG_SLOPE = 0.2

NODE_BLK = 1000
EDGE_BLK = 4000

N_WORKERS = 32
PER_CORE = N_EDGES // 2
PER_W = N_EDGES // N_WORKERS
C = 40
N_CHUNK = PER_W // C
NBUF = 2
N_PAD = 10240
RPT = N_PAD // 16
LANES = 16


N_NODE_BLKS = N_NODES // NODE_BLK


def _tc_front_body(ea_ref, w3_ref, batt_ref, x_ref, wfc_ref, bfc_ref,
                   w1_ref, w2_ref, e_ref, a_ref, tbh_ref):
    i = pl.program_id(0)
    e = jnp.dot(ea_ref[...], w3_ref[...], preferred_element_type=jnp.float32)
    e_ref[...] = e + batt_ref[...]

    @pl.when(i < N_NODE_BLKS)
    def _():
        h = jnp.dot(x_ref[...], wfc_ref[...], preferred_element_type=jnp.float32)
        h = h + bfc_ref[...]
        a_ref[...] = jnp.dot(h, w1_ref[...], preferred_element_type=jnp.float32)
        b = jnp.dot(h, w2_ref[...], preferred_element_type=jnp.float32)
        tbh_ref[...] = jnp.concatenate([b, h], axis=1)


def _node_blk_map(i):
    return (jnp.minimum(i, N_NODE_BLKS - 1), 0)


_tc_front = pl.pallas_call(
    _tc_front_body,
    grid=(N_EDGES // EDGE_BLK,),
    in_specs=[
        pl.BlockSpec((EDGE_BLK, D_EDGE), lambda i: (i, 0)),
        pl.BlockSpec((D_EDGE, D), lambda i: (0, 0)),
        pl.BlockSpec((1, D), lambda i: (0, 0)),
        pl.BlockSpec((NODE_BLK, D), _node_blk_map),
        pl.BlockSpec((D, D), lambda i: (0, 0)),
        pl.BlockSpec((1, D), lambda i: (0, 0)),
        pl.BlockSpec((D, D), lambda i: (0, 0)),
        pl.BlockSpec((D, D), lambda i: (0, 0)),
    ],
    out_specs=[
        pl.BlockSpec((EDGE_BLK, D), lambda i: (i, 0)),
        pl.BlockSpec((NODE_BLK, D), _node_blk_map),
        pl.BlockSpec((NODE_BLK, 2 * D), _node_blk_map),
    ],
    out_shape=[
        jax.ShapeDtypeStruct((N_EDGES, D), jnp.float32),
        jax.ShapeDtypeStruct((N_NODES, D), jnp.float32),
        jax.ShapeDtypeStruct((N_NODES, 2 * D), jnp.float32),
    ],
)


def _tc_sum_body(p_ref, o_ref):
    o_ref[...] = p_ref[0] + p_ref[1]


_tc_sum = pl.pallas_call(
    _tc_sum_body,
    grid=(N_NODES // NODE_BLK,),
    in_specs=[pl.BlockSpec((2, NODE_BLK, D), lambda i: (0, i, 0))],

    out_specs=pl.BlockSpec((NODE_BLK, D), lambda i: (i, 0)),
    out_shape=jax.ShapeDtypeStruct((N_NODES, D), jnp.float32),
)


_sc_mesh = plsc.VectorSubcoreMesh(core_axis_name="c", subcore_axis_name="s")


@functools.partial(
    pl.kernel,
    mesh=_sc_mesh,
    out_type=jax.ShapeDtypeStruct((2, N_PAD, D), jnp.float32),
    scratch_types=[
        pltpu.VMEM((NBUF, 2, C), jnp.int32),
        pltpu.VMEM((NBUF, C, D), jnp.float32),
        pltpu.VMEM((NBUF, C, 2 * D), jnp.float32),
        pltpu.VMEM((NBUF, C, D), jnp.float32),
        pltpu.VMEM_SHARED((N_PAD, D), jnp.float32),
        pltpu.SemaphoreType.DMA,
        pltpu.SemaphoreType.DMA,
    ],
)
def _sc_edges(a_hbm, tbh_hbm, e_hbm, rc_hbm, zero_hbm, out_hbm,
              idx, ga, gtbh, ge, acc, sem0, sem1):
    c = lax.axis_index("c")
    s = lax.axis_index("s")
    sems = (sem0, sem1)

    rowbase = s * RPT
    pltpu.sync_copy(zero_hbm.at[pl.ds(rowbase, RPT)],
                    acc.at[pl.ds(rowbase, RPT)])

    ebase = c * PER_CORE + s * PER_W
    cbase = ebase // C

    def _issue(b, k):
        pltpu.sync_copy(rc_hbm.at[cbase + k], idx.at[b])
        pltpu.async_copy(a_hbm.at[idx.at[b, 0]], ga.at[b], sems[b])
        pltpu.async_copy(tbh_hbm.at[idx.at[b, 1]], gtbh.at[b], sems[b])
        pltpu.async_copy(e_hbm.at[pl.ds(ebase + k * C, C)], ge.at[b], sems[b])

    for b in range(NBUF):
        _issue(b, b)
    plsc.subcore_barrier()

    def _pair(t, carry):
        for b in range(NBUF):
            k = t * NBUF + b
            off = ebase + k * C
            pltpu.make_async_copy(a_hbm.at[idx.at[b, 0]], ga.at[b], sems[b]).wait()
            pltpu.make_async_copy(tbh_hbm.at[idx.at[b, 1]], gtbh.at[b], sems[b]).wait()
            pltpu.make_async_copy(e_hbm.at[pl.ds(off, C)], ge.at[b], sems[b]).wait()

            @plsc.parallel_loop(0, C, unroll=8)
            def _edge(i):
                for j in range(D // LANES):
                    sl = pl.ds(j * LANES, LANES)
                    sv = ga[b, i, sl] + gtbh[b, i, sl] + ge[b, i, sl]
                    alpha = jnp.maximum(sv, sv * jnp.float32(NEG_SLOPE))
                    ga[b, i, sl] = gtbh[b, i, pl.ds(D + j * LANES, LANES)] * alpha

            pltpu.sync_copy(ga.at[b], acc.at[idx.at[b, 0]], add=True)

            @pl.when(k + NBUF < N_CHUNK)
            def _():
                _issue(b, k + NBUF)
        return carry

    lax.fori_loop(0, N_CHUNK // NBUF, _pair, 0)
    plsc.subcore_barrier()

    pltpu.sync_copy(acc.at[pl.ds(rowbase, RPT)],
                    out_hbm.at[c, pl.ds(rowbase, RPT)])


@jax.jit
def kernel(x, edge_index, edge_attr, W_fc, b_fc, W_att, b_att):
    row = edge_index[0].astype(jnp.int32)
    col = edge_index[1].astype(jnp.int32)
    W1 = W_att[:D]
    W2 = W_att[D:2 * D]
    W3 = W_att[2 * D:]
    E, A, TBH = _tc_front(edge_attr, W3, b_att.reshape(1, D),
                          x, W_fc, b_fc.reshape(1, D), W1, W2)
    zeros = jnp.zeros((N_PAD, D), jnp.float32)
    rc = jnp.stack([row.reshape(-1, C), col.reshape(-1, C)], axis=1)
    partials = _sc_edges(A, TBH, E, rc, zeros)
    return _tc_sum(partials)
